# baseline (speedup 1.0000x reference)
import functools
import math

import jax
import jax.numpy as jnp
from jax.experimental import pallas as pl
from jax.experimental.pallas import tpu as pltpu

D = 1024
NH = 16
HD = 64
DFF = 4096
VOCAB = 16384
NREC = 3
LPS = 2
EPS = 1e-5
ROPE_BASE = 10000.0
CAPS = [1.0, 2.0 / 3.0, 1.0 / 3.0]
AUXW = 0.001
ZW = 0.001


def _bdot(a, b, trans_b=False):
    dn = (((1,), (1,)), ((), ())) if trans_b else (((1,), (0,)), ((), ()))
    return jax.lax.dot_general(a.astype(jnp.bfloat16), b.astype(jnp.bfloat16),
                               dn, preferred_element_type=jnp.float32)


def _gather_body(sidx_ref, src_ref, out_ref):
    out_ref[...] = src_ref[...]


def _emb_gather(idx_flat, tok_emb):
    n = idx_flat.shape[0]
    grid_spec = pltpu.PrefetchScalarGridSpec(
        num_scalar_prefetch=1,
        grid=(n,),
        in_specs=[pl.BlockSpec((1, 1, D), lambda i, sidx: (sidx[i], 0, 0))],
        out_specs=pl.BlockSpec((1, 1, D), lambda i, sidx: (i, 0, 0)),
    )
    out = pl.pallas_call(
        _gather_body,
        grid_spec=grid_spec,
        out_shape=jax.ShapeDtypeStruct((n, 1, D), jnp.float32),
    )(idx_flat.astype(jnp.int32), tok_emb.reshape(VOCAB, 1, D))
    return out.reshape(n, D)


def _router_body(x_ref, w_ref, sc_ref, zl_ref, l1m_ref, dlog_ref):
    i = pl.program_id(0)
    rl = _bdot(x_ref[...], w_ref[...])
    sc = jax.nn.sigmoid(rl)
    sc_ref[...] = sc
    zl = jnp.sum(rl * rl).reshape(1, 1)
    l1m = jnp.sum(jnp.log(1.0 - sc + 1e-9)).reshape(1, 1)
    dlog = jnp.sum(jnp.log(sc + 1e-9) - jnp.log(1.0 - sc + 1e-9)).reshape(1, 1)

    @pl.when(i == 0)
    def _init():
        zl_ref[...] = zl
        l1m_ref[...] = l1m
        dlog_ref[...] = dlog

    @pl.when(i != 0)
    def _acc():
        zl_ref[...] += zl
        l1m_ref[...] += l1m
        dlog_ref[...] += dlog


def _router(x2d, rw):
    n = x2d.shape[0]
    tile = 1024
    scal = jax.ShapeDtypeStruct((1, 1), jnp.float32)
    return pl.pallas_call(
        _router_body,
        grid=(n // tile,),
        in_specs=[
            pl.BlockSpec((tile, D), lambda i: (i, 0)),
            pl.BlockSpec((D, 1), lambda i: (0, 0)),
        ],
        out_specs=[
            pl.BlockSpec((tile, 1), lambda i: (i, 0)),
            pl.BlockSpec((1, 1), lambda i: (0, 0)),
            pl.BlockSpec((1, 1), lambda i: (0, 0)),
            pl.BlockSpec((1, 1), lambda i: (0, 0)),
        ],
        out_shape=[jax.ShapeDtypeStruct((n, 1), jnp.float32), scal, scal, scal],
    )(x2d, rw)


def _tok_gather_body(kk, kkp, sti_ref, x_ref, sc_ref, xs_ref, g_ref, sel_ref):
    pid = pl.program_id(0)
    rloc = jax.lax.rem(pid, kkp)
    valid = rloc < kk
    vf = jnp.where(valid, 1.0, 0.0)
    scv = sc_ref[...]
    xs_ref[...] = x_ref[...] * vf
    g_ref[...] = scv * vf
    dlog = (jnp.where(valid,
                      jnp.log(scv + 1e-9) - jnp.log(1.0 - scv + 1e-9), 0.0)
            .reshape(1, 1))

    @pl.when(pid == 0)
    def _init():
        sel_ref[...] = dlog

    @pl.when(pid != 0)
    def _acc():
        sel_ref[...] += dlog


def _tok_gather(ti_flat, x2d, sc2d, b, t, kk, kkp):
    def src_map(i, sti):
        bb = i // kkp
        rloc = i - bb * kkp
        safe = jnp.minimum(rloc, kk - 1)
        return (bb * t + sti[bb * kk + safe], 0, 0)

    grid_spec = pltpu.PrefetchScalarGridSpec(
        num_scalar_prefetch=1,
        grid=(b * kkp,),
        in_specs=[
            pl.BlockSpec((1, 1, D), src_map),
            pl.BlockSpec((1, 1, 1), src_map),
        ],
        out_specs=[
            pl.BlockSpec((1, 1, D), lambda i, sti: (i, 0, 0)),
            pl.BlockSpec((1, 1, 1), lambda i, sti: (i, 0, 0)),
            pl.BlockSpec((1, 1), lambda i, sti: (0, 0)),
        ],
    )
    xs, g, sel = pl.pallas_call(
        functools.partial(_tok_gather_body, kk, kkp),
        grid_spec=grid_spec,
        out_shape=[
            jax.ShapeDtypeStruct((b * kkp, 1, D), jnp.float32),
            jax.ShapeDtypeStruct((b * kkp, 1, 1), jnp.float32),
            jax.ShapeDtypeStruct((1, 1), jnp.float32),
        ],
    )(ti_flat, x2d.reshape(b * t, 1, D), sc2d.reshape(b * t, 1, 1))
    return xs.reshape(b, kkp, D), g.reshape(b, kkp, 1), sel


def _scatter_body(sti_ref, src_ref, xin_ref, out_ref):
    out_ref[...] = src_ref[...]


def _scatter(ti_flat, outs2d, x2d, b, t, kk, kkp):
    grid_spec = pltpu.PrefetchScalarGridSpec(
        num_scalar_prefetch=1,
        grid=(b * kk,),
        in_specs=[
            pl.BlockSpec((1, 1, D), lambda i, sti: (
                (i // kk) * kkp + (i - (i // kk) * kk), 0, 0)),
            pl.BlockSpec(memory_space=pl.ANY),
        ],
        out_specs=pl.BlockSpec((1, 1, D), lambda i, sti: (
            (i // kk) * t + sti[i], 0, 0)),
    )
    out = pl.pallas_call(
        _scatter_body,
        grid_spec=grid_spec,
        out_shape=jax.ShapeDtypeStruct((b * t, 1, D), jnp.float32),
        input_output_aliases={2: 0},
    )(ti_flat, outs2d.reshape(-1, 1, D), x2d.reshape(b * t, 1, D))
    return out.reshape(b * t, D)


def _qkv_body(x_ref, ms_ref, n1_ref, w_ref, out_ref, nrm_ref):
    j = pl.program_id(2)

    @pl.when(j == 0)
    def _norm():
        xx = x_ref[0]
        nrm_ref[...] = xx * n1_ref[...] / jnp.sqrt(ms_ref[0] + EPS)

    out_ref[0] = _bdot(nrm_ref[...], w_ref[...])


def _qkv(xsp, ms, n1, wqkv):
    b, kkp, _ = xsp.shape
    return pl.pallas_call(
        _qkv_body,
        grid=(b, kkp // 128, 3),
        in_specs=[
            pl.BlockSpec((1, 128, D), lambda bb, i, j: (bb, i, 0)),
            pl.BlockSpec((1, 128, 1), lambda bb, i, j: (bb, i, 0)),
            pl.BlockSpec((1, D), lambda bb, i, j: (0, 0)),
            pl.BlockSpec((D, D), lambda bb, i, j: (0, j)),
        ],
        out_specs=pl.BlockSpec((1, 128, D), lambda bb, i, j: (bb, i, j)),
        out_shape=jax.ShapeDtypeStruct((b, kkp, 3 * D), jnp.float32),
        scratch_shapes=[pltpu.VMEM((128, D), jnp.float32)],
    )(xsp, ms, n1.reshape(1, D), wqkv)


def _rope_cs(x, c, s):
    x1 = x[:, : HD // 2]
    x2 = x[:, HD // 2:]
    return jnp.concatenate([x1 * c - x2 * s, x1 * s + x2 * c], axis=1)


def _flash_body(kkp, q_ref, k_ref, v_ref, cq_ref, sq_ref, ck_ref, sk_ref,
                o_ref, kr_ref):
    i = pl.program_id(1)

    @pl.when(i == 0)
    def _prep_k():
        kr_ref[...] = _rope_cs(k_ref[0], ck_ref[0], sk_ref[0])

    q = _rope_cs(q_ref[0], cq_ref[0], sq_ref[0])
    sc = _bdot(q, kr_ref[...], trans_b=True) / math.sqrt(HD)
    qi = jax.lax.broadcasted_iota(jnp.int32, sc.shape, 0) + i * 128
    ki = jax.lax.broadcasted_iota(jnp.int32, sc.shape, 1)
    sc = jnp.where(ki <= qi, sc, -1e9)
    m = jnp.max(sc, axis=1, keepdims=True)
    p = jnp.exp(sc - m)
    l = jnp.sum(p, axis=1, keepdims=True)
    o_ref[0] = _bdot(p / l, v_ref[0])


def _flash(q, k, v, cs, sn):
    bh, kkp, _ = q.shape
    return pl.pallas_call(
        functools.partial(_flash_body, kkp),
        grid=(bh, kkp // 128),
        in_specs=[
            pl.BlockSpec((1, 128, HD), lambda h, i: (h, i, 0)),
            pl.BlockSpec((1, kkp, HD), lambda h, i: (h, 0, 0)),
            pl.BlockSpec((1, kkp, HD), lambda h, i: (h, 0, 0)),
            pl.BlockSpec((1, 128, HD // 2), lambda h, i: (h // NH, i, 0)),
            pl.BlockSpec((1, 128, HD // 2), lambda h, i: (h // NH, i, 0)),
            pl.BlockSpec((1, kkp, HD // 2), lambda h, i: (h // NH, 0, 0)),
            pl.BlockSpec((1, kkp, HD // 2), lambda h, i: (h // NH, 0, 0)),
        ],
        out_specs=pl.BlockSpec((1, 128, HD), lambda h, i: (h, i, 0)),
        out_shape=jax.ShapeDtypeStruct((bh, kkp, HD), jnp.float32),
        scratch_shapes=[pltpu.VMEM((kkp, HD), jnp.float32)],
    )(q, k, v, cs, sn, cs, sn)


def _proj_body(o_ref, wo_ref, xs_ref, h_ref):
    h_ref[0] = xs_ref[0] + _bdot(o_ref[0], wo_ref[...])


def _proj(o, wo, xsp):
    b, kkp, _ = o.shape
    return pl.pallas_call(
        _proj_body,
        grid=(b, kkp // 128),
        in_specs=[
            pl.BlockSpec((1, 128, D), lambda bb, i: (bb, i, 0)),
            pl.BlockSpec((D, D), lambda bb, i: (0, 0)),
            pl.BlockSpec((1, 128, D), lambda bb, i: (bb, i, 0)),
        ],
        out_specs=pl.BlockSpec((1, 128, D), lambda bb, i: (bb, i, 0)),
        out_shape=jax.ShapeDtypeStruct((b, kkp, D), jnp.float32),
    )(o, wo, xsp)


def _ffn_body(gated, h_ref, ms_ref, n2_ref, w1_ref, w3_ref, w2_ref, g_ref,
              xs_ref, out_ref):
    hh = h_ref[0]
    nrm = hh * n2_ref[...] / jnp.sqrt(ms_ref[0] + EPS)
    a = jax.nn.silu(_bdot(nrm, w1_ref[...])) * _bdot(nrm, w3_ref[...])
    hfin = hh + _bdot(a, w2_ref[...])
    if gated:
        g = g_ref[0]
        out_ref[0] = g * hfin + (1.0 - g) * xs_ref[0]
    else:
        out_ref[0] = hfin


def _ffn(h, ms, n2, w1b, w3b, w2b, gp, xsp, gated):
    b, kkp, _ = h.shape
    return pl.pallas_call(
        functools.partial(_ffn_body, gated),
        grid=(b, kkp // 128),
        in_specs=[
            pl.BlockSpec((1, 128, D), lambda bb, i: (bb, i, 0)),
            pl.BlockSpec((1, 128, 1), lambda bb, i: (bb, i, 0)),
            pl.BlockSpec((1, D), lambda bb, i: (0, 0)),
            pl.BlockSpec((D, DFF), lambda bb, i: (0, 0)),
            pl.BlockSpec((D, DFF), lambda bb, i: (0, 0)),
            pl.BlockSpec((DFF, D), lambda bb, i: (0, 0)),
            pl.BlockSpec((1, 128, 1), lambda bb, i: (bb, i, 0)),
            pl.BlockSpec((1, 128, D), lambda bb, i: (bb, i, 0)),
        ],
        out_specs=pl.BlockSpec((1, 128, D), lambda bb, i: (bb, i, 0)),
        out_shape=jax.ShapeDtypeStruct((b, kkp, D), jnp.float32),
    )(h, ms, n2.reshape(1, D), w1b, w3b, w2b, gp, xsp)


def _logits_body(nv, x_ref, ms_ref, fn_ref, emb_ref, t_ref, lg_ref, loss_ref,
                 m_ref, l_ref, tv_ref):
    j = pl.program_id(0)
    i = pl.program_id(1)
    xx = x_ref[...]
    xn = xx * fn_ref[...] / jnp.sqrt(ms_ref[...] + EPS)
    lg = _bdot(xn, emb_ref[...], trans_b=True)
    lg_ref[...] = lg
    rm = jnp.max(lg, axis=1, keepdims=True)
    vt = lg.shape[1]
    tloc = t_ref[...] - j * vt
    oh = jax.lax.broadcasted_iota(jnp.int32, lg.shape, 1) == tloc
    pick = jnp.sum(jnp.where(oh, lg, 0.0), axis=1, keepdims=True)
    rows = pl.ds(i * 128, 128)

    @pl.when(j == 0)
    def _init():
        m_ref[rows, :] = rm
        l_ref[rows, :] = jnp.sum(jnp.exp(lg - rm), axis=1, keepdims=True)
        tv_ref[rows, :] = pick

    @pl.when(j != 0)
    def _upd():
        m_old = m_ref[rows, :]
        m_new = jnp.maximum(m_old, rm)
        l_ref[rows, :] = (l_ref[rows, :] * jnp.exp(m_old - m_new)
                          + jnp.sum(jnp.exp(lg - m_new), axis=1, keepdims=True))
        m_ref[rows, :] = m_new
        tv_ref[rows, :] += pick

    @pl.when(j == nv - 1)
    def _fin():
        lse = m_ref[rows, :] + jnp.log(l_ref[rows, :])
        contrib = jnp.sum(lse - tv_ref[rows, :]).reshape(1, 1)

        @pl.when(i == 0)
        def _first():
            loss_ref[...] = contrib

        @pl.when(i != 0)
        def _acc():
            loss_ref[...] += contrib


def _logits_loss(x2d, ms, fn, emb, t2d):
    n = x2d.shape[0]
    vt = 2048
    nv = VOCAB // vt
    nr = n // 128
    return pl.pallas_call(
        functools.partial(_logits_body, nv),
        grid=(nv, nr),
        in_specs=[
            pl.BlockSpec((128, D), lambda j, i: (i, 0)),
            pl.BlockSpec((128, 1), lambda j, i: (i, 0)),
            pl.BlockSpec((1, D), lambda j, i: (0, 0)),
            pl.BlockSpec((vt, D), lambda j, i: (j, 0)),
            pl.BlockSpec((128, 1), lambda j, i: (i, 0)),
        ],
        out_specs=[
            pl.BlockSpec((128, vt), lambda j, i: (i, j)),
            pl.BlockSpec((1, 1), lambda j, i: (0, 0)),
        ],
        out_shape=[
            jax.ShapeDtypeStruct((n, VOCAB), jnp.float32),
            jax.ShapeDtypeStruct((1, 1), jnp.float32),
        ],
        scratch_shapes=[pltpu.VMEM((n, 1), jnp.float32),
                        pltpu.VMEM((n, 1), jnp.float32),
                        pltpu.VMEM((n, 1), jnp.float32)],
    )(x2d, ms, fn.reshape(1, D), emb, t2d)


def _rope_tables(ps, kkp):
    b, klen = ps.shape
    inv = 1.0 / (ROPE_BASE ** (jnp.arange(HD // 2, dtype=jnp.float32)
                               / (HD // 2)))
    ang = ps[:, None, :, None] * inv[None, None, None, :]
    c = jnp.cos(ang).reshape(b, klen, HD // 2)
    s = jnp.sin(ang).reshape(b, klen, HD // 2)
    if klen < kkp:
        c = jnp.pad(c, ((0, 0), (0, kkp - klen), (0, 0)))
        s = jnp.pad(s, ((0, 0), (0, kkp - klen), (0, 0)))
    return c, s


def _transformer_block(h, p, wqkv, cs, sn):
    b, kkp, _ = h.shape
    ms = jnp.mean(h * h, axis=-1, keepdims=True)
    qkv = _qkv(h, ms, p['n1'], wqkv)
    qkv = qkv.reshape(b, kkp, 3, NH, HD).transpose(2, 0, 3, 1, 4)
    q = qkv[0].reshape(b * NH, kkp, HD)
    k = qkv[1].reshape(b * NH, kkp, HD)
    v = qkv[2].reshape(b * NH, kkp, HD)
    o = _flash(q, k, v, cs, sn)
    o = o.reshape(b, NH, kkp, HD).transpose(0, 2, 1, 3).reshape(b, kkp, D)
    return _proj(o, p['wo'], h)


def _apply_blocks(h, blocks, wqkv, w123b, cs, sn, gp, xsp):
    for bi in range(LPS):
        p = blocks[bi]
        w1b, w3b, w2b = w123b[bi]
        h = _transformer_block(h, p, wqkv[bi], cs, sn)
        ms2 = jnp.mean(h * h, axis=-1, keepdims=True)
        h = _ffn(h, ms2, p['n2'], w1b, w3b, w2b, gp, xsp,
                 gated=(bi == LPS - 1))
    return h


def kernel(idx, targets, params):
    b, t = idx.shape
    n = b * t
    tok_emb = params['tok_emb']
    blocks = params['blocks']
    wqkv = [jnp.concatenate([bl['wq'], bl['wk'], bl['wv']], axis=1)
            for bl in blocks]
    w123b = [(bl['w1'].astype(jnp.bfloat16), bl['w3'].astype(jnp.bfloat16),
              bl['w2'].astype(jnp.bfloat16)) for bl in blocks]

    x2d = _emb_gather(idx.reshape(-1), tok_emb)

    aux_sum = jnp.float32(0.0)
    zl_sum = jnp.float32(0.0)
    for r in range(NREC):
        kk = max(1, int(math.ceil(CAPS[r] * t)))
        kkp = ((kk + 127) // 128) * 128
        sc2d, zl_s, l1m_s, dlog_all = _router(x2d, params['router_w'][r])
        if kk == t:
            xsp = x2d.reshape(b, t, D)
            gp = sc2d.reshape(b, t, 1)
            ps = jnp.broadcast_to(
                jnp.arange(t, dtype=jnp.float32)[None, :], (b, t))
            sel_s = dlog_all
            ti_flat = None
        else:
            sc = sc2d.reshape(b, t)
            _, ti = jax.lax.top_k(sc, kk)
            ti = jnp.sort(ti, axis=-1)
            ti_flat = ti.reshape(-1).astype(jnp.int32)
            xsp, gp, sel_s = _tok_gather(ti_flat, x2d, sc2d, b, t, kk, kkp)
            ps = ti.astype(jnp.float32)

        cs, sn = _rope_tables(ps, kkp)
        h = _apply_blocks(xsp, blocks, wqkv, w123b, cs, sn, gp, xsp)

        if kk == t:
            x2d = h.reshape(n, D)
        else:
            x2d = _scatter(ti_flat, h.reshape(b * kkp, D), x2d, b, t, kk, kkp)

        aux_sum = aux_sum + (-(l1m_s[0, 0] + sel_s[0, 0]) / n)
        zl_sum = zl_sum + zl_s[0, 0] / n

    ms_f = jnp.mean(x2d * x2d, axis=-1, keepdims=True)
    logits2d, loss_s = _logits_loss(x2d, ms_f, params['final_norm'], tok_emb,
                                    targets.reshape(-1, 1).astype(jnp.int32))
    lm = loss_s[0, 0] / n
    aux = AUXW * aux_sum
    zl = ZW * zl_sum
    loss = lm + aux + zl
    return logits2d.reshape(b, t, VOCAB), loss, lm, aux, zl
